# baseline (speedup 1.0000x reference)
import jax
import jax.numpy as jnp
from jax.experimental import pallas as pl
from jax.experimental.pallas import tpu as pltpu

_NUM_EMB = 1024
_EMB_DIM = 64
_BETA = 0.25
_EPS = 1e-10


def _vq_kernel(z_ref, emb_ref, enc_ref, zq_ref, ssq_ref, counts_ref,
               counts_acc, ssq_acc):
    i = pl.program_id(0)
    nsteps = pl.num_programs(0)

    @pl.when(i == 0)
    def _init():
        counts_acc[...] = jnp.zeros_like(counts_acc)
        ssq_acc[0, 0] = 0.0

    x = z_ref[0]
    xt = x.T
    emb = emb_ref[...]

    inner = jax.lax.dot_general(
        xt, emb, (((1,), (1,)), ((), ())),
        preferred_element_type=jnp.float32)
    flat_l2 = jnp.sum(xt * xt, axis=1, keepdims=True)
    emb_l2 = jnp.sum(emb * emb, axis=1)[None, :]
    dist = (flat_l2 + emb_l2) - 2.0 * inner
    iota = jax.lax.broadcasted_iota(
        jnp.int32, (xt.shape[0], _NUM_EMB), 1).astype(jnp.float32)
    dmin = jnp.min(dist, axis=1, keepdims=True)
    masked = jnp.where(dist == dmin, iota, jnp.float32(2.0 ** 30))
    idxf = jnp.min(masked, axis=1, keepdims=True)
    onehot = (masked == idxf).astype(jnp.float32)
    enc_ref[...] = onehot

    zq_rows = jax.lax.dot_general(
        onehot.astype(jnp.bfloat16), emb.astype(jnp.bfloat16),
        (((1,), (0,)), ((), ())),
        preferred_element_type=jnp.float32)
    zq_ref[0] = zq_rows.T.reshape(z_ref.shape[1], 32, 32)

    diff = zq_rows - xt
    ssq_acc[0, 0] += jnp.sum(diff * diff)
    counts_acc[...] += jnp.sum(onehot, axis=0, keepdims=True)

    @pl.when(i == nsteps - 1)
    def _fin():
        ssq_ref[...] = jnp.full((1, 1), ssq_acc[0, 0], jnp.float32)
        counts_ref[...] = counts_acc[...]


def _vq_shard(z4, embedding):
    b, c, h, w = z4.shape
    return pl.pallas_call(
        _vq_kernel,
        grid=(b,),
        in_specs=[
            pl.BlockSpec((1, c, h * w), lambda i: (i, 0, 0)),
            pl.BlockSpec((_NUM_EMB, _EMB_DIM), lambda i: (0, 0)),
        ],
        out_specs=[
            pl.BlockSpec((h * w, _NUM_EMB), lambda i: (i, 0)),
            pl.BlockSpec((1, c, h, w), lambda i: (i, 0, 0, 0)),
            pl.BlockSpec((1, 1), lambda i: (0, 0)),
            pl.BlockSpec((1, _NUM_EMB), lambda i: (0, 0)),
        ],
        out_shape=[
            jax.ShapeDtypeStruct((b * h * w, _NUM_EMB), jnp.float32),
            jax.ShapeDtypeStruct((b, c, h, w), jnp.float32),
            jax.ShapeDtypeStruct((1, 1), jnp.float32),
            jax.ShapeDtypeStruct((1, _NUM_EMB), jnp.float32),
        ],
        scratch_shapes=[
            pltpu.VMEM((1, _NUM_EMB), jnp.float32),
            pltpu.SMEM((1, 1), jnp.float32),
        ],
    )(z4.reshape(b, c, h * w), embedding)


def kernel(z_e, embedding):
    B, C, H, W = z_e.shape
    HW = H * W
    enc, zq, ssq, counts = _vq_shard(z_e, embedding)
    n_vec = B * HW
    mean_sq = jnp.sum(ssq) / (n_vec * _EMB_DIM)
    vq_loss = _BETA * mean_sq + mean_sq
    p = jnp.sum(counts, axis=0) * (1.0 / n_vec)
    perplexity = jnp.exp(-jnp.sum(p * jnp.log(p + _EPS)))
    return (vq_loss, zq, perplexity, enc)


# speedup vs baseline: 1.2028x; 1.2028x over previous
import jax
import jax.numpy as jnp
from jax.experimental import pallas as pl
from jax.experimental.pallas import tpu as pltpu

_NUM_EMB = 1024
_EMB_DIM = 64
_BETA = 0.25
_EPS = 1e-10


def _vq_kernel(z_ref, emb_ref, enc_ref, zq_ref, loss_ref, perp_ref,
               counts_acc, ssq_acc):
    i = pl.program_id(0)
    nsteps = pl.num_programs(0)

    @pl.when(i == 0)
    def _init():
        counts_acc[...] = jnp.zeros_like(counts_acc)
        ssq_acc[0, 0] = 0.0

    x = z_ref[0]
    xt = x.T
    emb = emb_ref[...]

    inner = jax.lax.dot_general(
        xt, emb, (((1,), (1,)), ((), ())),
        preferred_element_type=jnp.float32)
    flat_l2 = jnp.sum(xt * xt, axis=1, keepdims=True)
    emb_l2 = jnp.sum(emb * emb, axis=1)[None, :]
    dist = (flat_l2 + emb_l2) - 2.0 * inner
    iota = jax.lax.broadcasted_iota(
        jnp.int32, (xt.shape[0], _NUM_EMB), 1).astype(jnp.float32)
    dmin = jnp.min(dist, axis=1, keepdims=True)
    masked = jnp.where(dist == dmin, iota, jnp.float32(2.0 ** 30))
    idxf = jnp.min(masked, axis=1, keepdims=True)
    onehot = (masked == idxf).astype(jnp.float32)
    enc_ref[...] = onehot

    zq_rows = jax.lax.dot_general(
        onehot.astype(jnp.bfloat16), emb.astype(jnp.bfloat16),
        (((1,), (0,)), ((), ())),
        preferred_element_type=jnp.float32)
    zq_ref[0] = zq_rows.T

    diff = zq_rows - xt
    ssq_acc[0, 0] += jnp.sum(diff * diff)
    counts_acc[...] += jnp.sum(onehot, axis=0, keepdims=True)

    @pl.when(i == nsteps - 1)
    def _fin():
        n_vec = nsteps * 1024
        mean_sq = ssq_acc[0, 0] / (n_vec * _EMB_DIM)
        loss_ref[...] = jnp.full((1, 1), _BETA * mean_sq + mean_sq, jnp.float32)
        p = counts_acc[...] * (1.0 / n_vec)
        plogp = p * jnp.log(p + _EPS)
        perp_ref[...] = jnp.exp(-jnp.sum(plogp, axis=1, keepdims=True))


def kernel(z_e, embedding):
    B, C, H, W = z_e.shape
    HW = H * W
    z3 = z_e.reshape(B, C, HW)
    enc, zq3, loss, perp = pl.pallas_call(
        _vq_kernel,
        grid=(B,),
        in_specs=[
            pl.BlockSpec((1, C, HW), lambda i: (i, 0, 0)),
            pl.BlockSpec((_NUM_EMB, _EMB_DIM), lambda i: (0, 0)),
        ],
        out_specs=[
            pl.BlockSpec((HW, _NUM_EMB), lambda i: (i, 0)),
            pl.BlockSpec((1, C, HW), lambda i: (i, 0, 0)),
            pl.BlockSpec((1, 1), lambda i: (0, 0)),
            pl.BlockSpec((1, 1), lambda i: (0, 0)),
        ],
        out_shape=[
            jax.ShapeDtypeStruct((B * HW, _NUM_EMB), jnp.float32),
            jax.ShapeDtypeStruct((B, C, HW), jnp.float32),
            jax.ShapeDtypeStruct((1, 1), jnp.float32),
            jax.ShapeDtypeStruct((1, 1), jnp.float32),
        ],
        scratch_shapes=[
            pltpu.VMEM((1, _NUM_EMB), jnp.float32),
            pltpu.SMEM((1, 1), jnp.float32),
        ],
    )(z3, embedding)
    zq = zq3.reshape(B, C, H, W)
    return (loss[0, 0], zq, perp[0, 0], enc)


# speedup vs baseline: 1.2616x; 1.0489x over previous
import jax
import jax.numpy as jnp
from jax.experimental import pallas as pl
from jax.experimental.pallas import tpu as pltpu

_NUM_EMB = 1024
_EMB_DIM = 64
_BETA = 0.25
_EPS = 1e-10


def _vq_kernel(z_ref, emb_ref, enc_ref, zq_ref, loss_ref, perp_ref,
               counts_acc, ssq_acc):
    i = pl.program_id(0)
    nsteps = pl.num_programs(0)

    @pl.when(i == 0)
    def _init():
        counts_acc[...] = jnp.zeros_like(counts_acc)
        ssq_acc[0, 0] = 0.0

    x = z_ref[...]
    npx = x.shape[0] * x.shape[2]
    xt = jnp.transpose(x, (0, 2, 1)).reshape(npx, x.shape[1])
    emb = emb_ref[...]

    inner2 = jax.lax.dot_general(
        xt, emb + emb, (((1,), (1,)), ((), ())),
        preferred_element_type=jnp.float32)
    flat_l2 = jnp.sum(xt * xt, axis=1, keepdims=True)
    emb_l2 = jnp.sum(emb * emb, axis=1)[None, :]
    dist = (flat_l2 + emb_l2) - inner2
    iota = jax.lax.broadcasted_iota(
        jnp.int32, (xt.shape[0], _NUM_EMB), 1).astype(jnp.float32)
    dmin = jnp.min(dist, axis=1, keepdims=True)
    masked = jnp.where(dist == dmin, iota, jnp.float32(2.0 ** 30))
    idxf = jnp.min(masked, axis=1, keepdims=True)
    onehot = (masked == idxf).astype(jnp.float32)
    enc_ref[...] = onehot

    zq_rows = jax.lax.dot_general(
        onehot.astype(jnp.bfloat16), emb.astype(jnp.bfloat16),
        (((1,), (0,)), ((), ())),
        preferred_element_type=jnp.float32)
    zq_ref[...] = jnp.transpose(
        zq_rows.reshape(x.shape[0], x.shape[2], x.shape[1]), (0, 2, 1))

    diff = zq_rows - xt
    ssq_acc[0, 0] += jnp.sum(diff * diff)
    counts_acc[...] += jnp.sum(onehot, axis=0, keepdims=True)

    @pl.when(i == nsteps - 1)
    def _fin():
        n_vec = nsteps * 2048
        mean_sq = ssq_acc[0, 0] / (n_vec * _EMB_DIM)
        loss_ref[...] = jnp.full((1, 1), _BETA * mean_sq + mean_sq, jnp.float32)
        p = counts_acc[...] * (1.0 / n_vec)
        plogp = p * jnp.log(p + _EPS)
        perp_ref[...] = jnp.exp(-jnp.sum(plogp, axis=1, keepdims=True))


def kernel(z_e, embedding):
    B, C, H, W = z_e.shape
    HW = H * W
    z3 = z_e.reshape(B, C, HW)
    enc, zq3, loss, perp = pl.pallas_call(
        _vq_kernel,
        grid=(B // 2,),
        in_specs=[
            pl.BlockSpec((2, C, HW), lambda i: (i, 0, 0)),
            pl.BlockSpec((_NUM_EMB, _EMB_DIM), lambda i: (0, 0)),
        ],
        out_specs=[
            pl.BlockSpec((2 * HW, _NUM_EMB), lambda i: (i, 0)),
            pl.BlockSpec((2, C, HW), lambda i: (i, 0, 0)),
            pl.BlockSpec((1, 1), lambda i: (0, 0)),
            pl.BlockSpec((1, 1), lambda i: (0, 0)),
        ],
        out_shape=[
            jax.ShapeDtypeStruct((B * HW, _NUM_EMB), jnp.float32),
            jax.ShapeDtypeStruct((B, C, HW), jnp.float32),
            jax.ShapeDtypeStruct((1, 1), jnp.float32),
            jax.ShapeDtypeStruct((1, 1), jnp.float32),
        ],
        scratch_shapes=[
            pltpu.VMEM((1, _NUM_EMB), jnp.float32),
            pltpu.SMEM((1, 1), jnp.float32),
        ],
    )(z3, embedding)
    zq = zq3.reshape(B, C, H, W)
    return (loss[0, 0], zq, perp[0, 0], enc)
